# baseline (speedup 1.0000x reference)
import functools

import jax
import jax.numpy as jnp
from jax import lax
from jax.experimental import pallas as pl
from jax.experimental.pallas import tpu as pltpu
from jax.experimental.pallas import tpu_sc as plsc

LAMBDA_L = 0.01
LAMBDA_S = 0.01
EPS = 1e-08

B, H, W = 8, 512, 512
NA = 64
NC, NS, L = 2, 16, 16
NW = NC * NS
ELEMS_PER_TILE = B * H * W // NW
ROWS_PER_TILE = B * H // NW
TILES_PER_BATCH = NW // B
UNROLL = 4

ROWS_BLK = 16
BLK = ROWS_BLK * W
NBLK = ROWS_PER_TILE // ROWS_BLK
GROUPS = BLK // (UNROLL * L)

_mesh = plsc.VectorSubcoreMesh(core_axis_name="c", subcore_axis_name="s")
_params = pltpu.CompilerParams(needs_layout_passes=False)


@functools.partial(
    pl.kernel,
    mesh=_mesh,
    compiler_params=_params,
    out_type=jax.ShapeDtypeStruct((B, 1, H, W), jnp.float32),
    scratch_types=[
        pltpu.VMEM((ROWS_BLK, W), jnp.float32),
        pltpu.VMEM((ROWS_BLK, W), jnp.float32),
        pltpu.VMEM((ROWS_BLK, W), jnp.int32),
        pltpu.VMEM((ROWS_BLK, W), jnp.float32),
        pltpu.VMEM((ROWS_BLK, W), jnp.float32),
        pltpu.VMEM((ROWS_BLK, W), jnp.int32),
        pltpu.VMEM((ELEMS_PER_TILE,), jnp.float32),
        pltpu.VMEM((L * NA,), jnp.float32),
        pltpu.VMEM((NA,), jnp.float32),
        pltpu.VMEM_SHARED((NS * NA,), jnp.float32),
        pltpu.VMEM((TILES_PER_BATCH * NA,), jnp.float32),
        pltpu.VMEM((NA,), jnp.float32),
        pltpu.VMEM((NA,), jnp.float32),
        pltpu.SemaphoreType.DMA,
        pltpu.SemaphoreType.DMA,
        pltpu.SemaphoreType.DMA,
        pltpu.SemaphoreType.DMA,
    ],
)
def _fused(l_hbm, s_hbm, a_hbm, c_hbm, out_hbm,
           lbuf0, sbuf0, abuf0, lbuf1, sbuf1, abuf1,
           score_res, accum, sums, shared, pbuf, cbuf, fbuf,
           sem0, sem1, osem0, osem1):
    cid = lax.axis_index("c")
    sid = lax.axis_index("s")
    wid = cid * NS + sid
    batch = wid // TILES_PER_BATCH
    row0 = (wid % TILES_PER_BATCH) * ROWS_PER_TILE
    lbufs, sbufs, abufs = (lbuf0, lbuf1), (sbuf0, sbuf1), (abuf0, abuf1)
    sems, osems = (sem0, sem1), (osem0, osem1)

    zero = jnp.zeros((L,), jnp.float32)
    for k in range(NA):
        accum[pl.ds(k * L, L)] = zero
    lane_iota = jnp.arange(L, dtype=jnp.int32)

    def issue_in(bi, slot):
        r = row0 + bi * ROWS_BLK
        return (
            pltpu.async_copy(l_hbm.at[batch, 0, pl.ds(r, ROWS_BLK), :],
                             lbufs[slot], sems[slot]),
            pltpu.async_copy(s_hbm.at[batch, 0, pl.ds(r, ROWS_BLK), :],
                             sbufs[slot], sems[slot]),
            pltpu.async_copy(a_hbm.at[batch, pl.ds(r, ROWS_BLK), :],
                             abufs[slot], sems[slot]),
        )

    pending = {0: issue_in(0, 0)}
    for bi in range(NBLK):
        slot = bi % 2
        if bi + 1 < NBLK:
            pending[(bi + 1) % 2] = issue_in(bi + 1, (bi + 1) % 2)
        for h in pending[slot]:
            h.wait()
        lbuf, sbuf, abuf = lbufs[slot], sbufs[slot], abufs[slot]

        def body_a(i, _):
            off = i * (UNROLL * L)
            r = off // W
            c = off % W
            for j in range(UNROLL):
                p = pl.ds(c + j * L, L)
                score = (lbuf[r, p] + LAMBDA_L) * (sbuf[r, p] + LAMBDA_S)
                score_res[pl.ds(bi * BLK + off + j * L, L)] = score
                plsc.addupdate_scatter(
                    accum, [abuf[r, p] * L + lane_iota], score)
            return 0

        lax.fori_loop(0, GROUPS, body_a, 0)

    stride_iota = lane_iota * L
    for k in range(NA // L):
        t = jnp.zeros((L,), jnp.float32)
        for lane in range(L):
            t = t + plsc.load_gather(accum, [stride_iota + (k * L * L + lane)])
        sums[pl.ds(k * L, L)] = t

    pltpu.sync_copy(sums, shared.at[pl.ds(sid * NA, NA)])
    plsc.subcore_barrier()
    lb = sid // TILES_PER_BATCH
    pltpu.sync_copy(shared.at[pl.ds(lb * TILES_PER_BATCH * NA,
                                    TILES_PER_BATCH * NA)], pbuf)
    pltpu.sync_copy(c_hbm, cbuf)
    for k in range(NA // L):
        t = pbuf[pl.ds(k * L, L)]
        for j in range(1, TILES_PER_BATCH):
            t = t + pbuf[pl.ds(j * NA + k * L, L)]
        fbuf[pl.ds(k * L, L)] = cbuf[pl.ds(k * L, L)] / (t + EPS)

    def issue_adm(bi, slot):
        r = row0 + bi * ROWS_BLK
        return (pltpu.async_copy(a_hbm.at[batch, pl.ds(r, ROWS_BLK), :],
                                 abufs[slot], sems[slot]),)

    pending = {0: issue_adm(0, 0)}
    out_pending = {0: None, 1: None}
    obufs = (lbuf0, lbuf1)
    for bi in range(NBLK):
        slot = bi % 2
        if bi + 1 < NBLK:
            pending[(bi + 1) % 2] = issue_adm(bi + 1, (bi + 1) % 2)
        for h in pending[slot]:
            h.wait()
        if out_pending[slot] is not None:
            out_pending[slot].wait()
        abuf, obuf = abufs[slot], obufs[slot]

        def body_b(i, _):
            off = i * (UNROLL * L)
            r = off // W
            c = off % W
            for j in range(UNROLL):
                p = pl.ds(c + j * L, L)
                score = score_res[pl.ds(bi * BLK + off + j * L, L)]
                f = plsc.load_gather(fbuf, [abuf[r, p]])
                obuf[r, p] = score * f
            return 0

        lax.fori_loop(0, GROUPS, body_b, 0)
        r = row0 + bi * ROWS_BLK
        out_pending[slot] = pltpu.async_copy(
            obuf, out_hbm.at[batch, 0, pl.ds(r, ROWS_BLK), :], osems[slot])
    for slot in (0, 1):
        if out_pending[slot] is not None:
            out_pending[slot].wait()


def kernel(lights, settlement, admin_ids, census_totals):
    return _fused(lights, settlement, admin_ids, census_totals)


# speedup vs baseline: 1.2816x; 1.2816x over previous
import functools

import jax
import jax.numpy as jnp
from jax import lax
from jax.experimental import pallas as pl
from jax.experimental.pallas import tpu as pltpu
from jax.experimental.pallas import tpu_sc as plsc

LAMBDA_L = 0.01
LAMBDA_S = 0.01
EPS = 1e-08

B, H, W = 8, 512, 512
NA = 64
NC, NS, L = 2, 16, 16
NW = NC * NS
ELEMS_PER_TILE = B * H * W // NW
ROWS_PER_TILE = B * H // NW
TILES_PER_BATCH = NW // B
UNROLL = 4

ROWS_BLK = 16
BLK = ROWS_BLK * W
NBLK = ROWS_PER_TILE // ROWS_BLK
GROUPS = BLK // (UNROLL * L)

_mesh = plsc.VectorSubcoreMesh(core_axis_name="c", subcore_axis_name="s")
_params = pltpu.CompilerParams(needs_layout_passes=False)


@functools.partial(
    pl.kernel,
    mesh=_mesh,
    compiler_params=_params,
    out_type=jax.ShapeDtypeStruct((B, 1, H, W), jnp.float32),
    scratch_types=[
        pltpu.VMEM((ROWS_BLK, W), jnp.float32),
        pltpu.VMEM((ROWS_BLK, W), jnp.float32),
        pltpu.VMEM((ROWS_BLK, W), jnp.int32),
        pltpu.VMEM((ROWS_BLK, W), jnp.float32),
        pltpu.VMEM((ROWS_BLK, W), jnp.float32),
        pltpu.VMEM((ROWS_BLK, W), jnp.int32),
        pltpu.VMEM((ROWS_BLK, W), jnp.float32),
        pltpu.VMEM((ROWS_BLK, W), jnp.float32),
        pltpu.VMEM((L * NA,), jnp.float32),
        pltpu.VMEM((NA,), jnp.float32),
        pltpu.VMEM_SHARED((NS * NA,), jnp.float32),
        pltpu.VMEM((TILES_PER_BATCH * NA,), jnp.float32),
        pltpu.VMEM((NA,), jnp.float32),
        pltpu.VMEM((NA,), jnp.float32),
        pltpu.SemaphoreType.DMA,
        pltpu.SemaphoreType.DMA,
        pltpu.SemaphoreType.DMA,
        pltpu.SemaphoreType.DMA,
    ],
)
def _fused(l_hbm, s_hbm, a_hbm, c_hbm, out_hbm,
           lbuf0, sbuf0, abuf0, lbuf1, sbuf1, abuf1,
           score_res0, score_res1, accum, sums, shared, pbuf, cbuf, fbuf,
           sem0, sem1, osem0, osem1):
    cid = lax.axis_index("c")
    sid = lax.axis_index("s")
    wid = cid * NS + sid
    batch = wid // TILES_PER_BATCH
    row0 = (wid % TILES_PER_BATCH) * ROWS_PER_TILE
    lbufs, sbufs, abufs = (lbuf0, lbuf1), (sbuf0, sbuf1), (abuf0, abuf1)
    sems, osems = (sem0, sem1), (osem0, osem1)

    zero = jnp.zeros((L,), jnp.float32)
    for k in range(NA):
        accum[pl.ds(k * L, L)] = zero
    lane_iota = jnp.arange(L, dtype=jnp.int32)

    def issue_in(bi, slot):
        r = row0 + bi * ROWS_BLK
        return (
            pltpu.async_copy(l_hbm.at[batch, 0, pl.ds(r, ROWS_BLK), :],
                             lbufs[slot], sems[slot]),
            pltpu.async_copy(s_hbm.at[batch, 0, pl.ds(r, ROWS_BLK), :],
                             sbufs[slot], sems[slot]),
            pltpu.async_copy(a_hbm.at[batch, pl.ds(r, ROWS_BLK), :],
                             abufs[slot], sems[slot]),
        )

    pending = {0: issue_in(0, 0)}
    for bi in range(NBLK):
        slot = bi % 2
        if bi + 1 < NBLK:
            pending[(bi + 1) % 2] = issue_in(bi + 1, (bi + 1) % 2)
        for h in pending[slot]:
            h.wait()
        lbuf, sbuf, abuf = lbufs[slot], sbufs[slot], abufs[slot]

        def body_a(i, _):
            off = i * (UNROLL * L)
            r = off // W
            c = off % W
            for j in range(UNROLL):
                p = pl.ds(c + j * L, L)
                score = (lbuf[r, p] + LAMBDA_L) * (sbuf[r, p] + LAMBDA_S)
                plsc.addupdate_scatter(
                    accum, [abuf[r, p] * L + lane_iota], score)
            return 0

        lax.fori_loop(0, GROUPS, body_a, 0)

    stride_iota = lane_iota * L
    for k in range(NA // L):
        t = jnp.zeros((L,), jnp.float32)
        for lane in range(L):
            t = t + plsc.load_gather(accum, [stride_iota + (k * L * L + lane)])
        sums[pl.ds(k * L, L)] = t

    pltpu.sync_copy(sums, shared.at[pl.ds(sid * NA, NA)])
    plsc.subcore_barrier()
    lb = sid // TILES_PER_BATCH
    pltpu.sync_copy(shared.at[pl.ds(lb * TILES_PER_BATCH * NA,
                                    TILES_PER_BATCH * NA)], pbuf)
    pltpu.sync_copy(c_hbm, cbuf)
    for k in range(NA // L):
        t = pbuf[pl.ds(k * L, L)]
        for j in range(1, TILES_PER_BATCH):
            t = t + pbuf[pl.ds(j * NA + k * L, L)]
        fbuf[pl.ds(k * L, L)] = cbuf[pl.ds(k * L, L)] / (t + EPS)

    pending = {0: issue_in(0, 0)}
    out_pending = {0: None, 1: None}
    obufs = (score_res0, score_res1)
    for bi in range(NBLK):
        slot = bi % 2
        if bi + 1 < NBLK:
            pending[(bi + 1) % 2] = issue_in(bi + 1, (bi + 1) % 2)
        for h in pending[slot]:
            h.wait()
        if out_pending[slot] is not None:
            out_pending[slot].wait()
        lbuf, sbuf, abuf, obuf = lbufs[slot], sbufs[slot], abufs[slot], obufs[slot]

        def body_b(i, _):
            off = i * (UNROLL * L)
            r = off // W
            c = off % W
            for j in range(UNROLL):
                p = pl.ds(c + j * L, L)
                score = (lbuf[r, p] + LAMBDA_L) * (sbuf[r, p] + LAMBDA_S)
                f = plsc.load_gather(fbuf, [abuf[r, p]])
                obuf[r, p] = score * f
            return 0

        lax.fori_loop(0, GROUPS, body_b, 0)
        r = row0 + bi * ROWS_BLK
        out_pending[slot] = pltpu.async_copy(
            obuf, out_hbm.at[batch, 0, pl.ds(r, ROWS_BLK), :], osems[slot])
    for slot in (0, 1):
        if out_pending[slot] is not None:
            out_pending[slot].wait()


def kernel(lights, settlement, admin_ids, census_totals):
    return _fused(lights, settlement, admin_ids, census_totals)
